# Initial kernel scaffold; baseline (speedup 1.0000x reference)
import jax
import jax.numpy as jnp
from jax.experimental import pallas as pl


def kernel(latent, seq, noise_u, W_sink, b_sink, W_mask, b_mask):
    raise NotImplementedError("write your pallas kernel here")


# speedup vs baseline: 4.2809x; 4.2809x over previous
import jax
import jax.numpy as jnp
from jax.experimental import pallas as pl
from jax.experimental.pallas import tpu as pltpu

_EPS = 1e-20
_N_ITERS = 5
_BLK = 1024


def _body(lat_ref, seq_ref, noise_ref, WsT_ref, bs_ref, WmT_ref, bm_ref,
          ord_ref, stop_ref):
    S = noise_ref.shape[0]
    K = WmT_ref.shape[0]
    D = seq_ref.shape[1] // K

    latT = jnp.swapaxes(lat_ref[...], 0, 1)

    laT = jnp.dot(WsT_ref[...], latT,
                  preferred_element_type=jnp.float32) + bs_ref[...]
    a0 = jnp.exp(laT)

    st = jnp.dot(WmT_ref[...], latT,
                 preferred_element_type=jnp.float32) + bm_ref[...]
    m = jnp.max(st, axis=0, keepdims=True)
    e = jnp.exp(st - m)
    stop_ref[...] = e / jnp.sum(e, axis=0, keepdims=True)

    seqT = jnp.swapaxes(seq_ref[...], 0, 1)
    seqr = [seqT[D * j:D * (j + 1)] for j in range(K)]

    for s in range(S):
        uT = jnp.swapaxes(noise_ref[s], 0, 1)
        denom = _EPS - jnp.log(uT + _EPS)
        alpha = a0 / denom
        rows = [alpha[K * r:K * (r + 1)] for r in range(K)]
        for _ in range(_N_ITERS):
            rows = [rr / jnp.sum(rr, axis=0, keepdims=True) for rr in rows]
            cs = rows[0]
            for rr in rows[1:]:
                cs = cs + rr
            rcs = 1.0 / cs
            rows = [rr * rcs for rr in rows]
        ords = []
        for i in range(K):
            acc = rows[0][i:i + 1] * seqr[0]
            for j in range(1, K):
                acc = acc + rows[j][i:i + 1] * seqr[j]
            ords.append(acc)
        ordT = jnp.concatenate(ords, axis=0)
        ord_ref[s] = jnp.swapaxes(ordT, 0, 1)


def kernel(latent, seq, noise_u, W_sink, b_sink, W_mask, b_mask):
    B, DL = latent.shape
    _, K, D = seq.shape
    S = noise_u.shape[0] // B
    KK = K * K
    blk = min(_BLK, B)

    noise3 = noise_u.reshape(S, B, KK)
    seq2 = seq.reshape(B, K * D)
    WsT = W_sink.T
    WmT = W_mask.T
    bs2 = b_sink.reshape(KK, 1)
    bm2 = b_mask.reshape(K, 1)

    ordered3, stoppingT = pl.pallas_call(
        _body,
        grid=(B // blk,),
        in_specs=[
            pl.BlockSpec((blk, DL), lambda i: (i, 0)),
            pl.BlockSpec((blk, K * D), lambda i: (i, 0)),
            pl.BlockSpec((S, blk, KK), lambda i: (0, i, 0)),
            pl.BlockSpec((KK, DL), lambda i: (0, 0)),
            pl.BlockSpec((KK, 1), lambda i: (0, 0)),
            pl.BlockSpec((K, DL), lambda i: (0, 0)),
            pl.BlockSpec((K, 1), lambda i: (0, 0)),
        ],
        out_specs=[
            pl.BlockSpec((S, blk, K * D), lambda i: (0, i, 0)),
            pl.BlockSpec((K, blk), lambda i: (0, i)),
        ],
        out_shape=[
            jax.ShapeDtypeStruct((S, B, K * D), jnp.float32),
            jax.ShapeDtypeStruct((K, B), jnp.float32),
        ],
        compiler_params=pltpu.CompilerParams(
            dimension_semantics=("parallel",),
        ),
        name="sinkhorn_net",
    )(latent, seq2, noise3, WsT, bs2, WmT, bm2)

    return ordered3.reshape(S * B, K, D), stoppingT.T


# speedup vs baseline: 5.8766x; 1.3727x over previous
import jax
import jax.numpy as jnp
from jax.experimental import pallas as pl
from jax.experimental.pallas import tpu as pltpu

_EPS = 1e-20
_N_ITERS = 5
_BLK = 1024


def _make_body(S, K, D):
    def _body(latT_ref, seqT_ref, *rest):
        noise_refs = rest[:S]
        WsT_ref, bs_ref, WmT_ref, bm_ref, ord_ref, stop_ref = rest[S:]

        latT = latT_ref[...]

        laT = jnp.dot(WsT_ref[...], latT,
                      preferred_element_type=jnp.float32) + bs_ref[...]
        a0 = jnp.exp(laT)

        st = jnp.dot(WmT_ref[...], latT,
                     preferred_element_type=jnp.float32) + bm_ref[...]
        m = jnp.max(st, axis=0, keepdims=True)
        e = jnp.exp(st - m)
        stop_ref[...] = e / jnp.sum(e, axis=0, keepdims=True)

        seqT = seqT_ref[...]
        seqr = [seqT[D * j:D * (j + 1)] for j in range(K)]

        for s in range(S):
            uT = noise_refs[s][...]
            denom = _EPS - jnp.log(uT + _EPS)
            alpha = a0 / denom
            rows = [alpha[K * r:K * (r + 1)] for r in range(K)]
            for _ in range(_N_ITERS):
                rows = [rr / jnp.sum(rr, axis=0, keepdims=True)
                        for rr in rows]
                cs = rows[0]
                for rr in rows[1:]:
                    cs = cs + rr
                rcs = 1.0 / cs
                rows = [rr * rcs for rr in rows]
            ords = []
            for i in range(K):
                acc = rows[0][i:i + 1] * seqr[0]
                for j in range(1, K):
                    acc = acc + rows[j][i:i + 1] * seqr[j]
                ords.append(acc)
            ord_ref[s] = jnp.concatenate(ords, axis=0)

    return _body


def kernel(latent, seq, noise_u, W_sink, b_sink, W_mask, b_mask):
    B, DL = latent.shape
    _, K, D = seq.shape
    S = noise_u.shape[0] // B
    KK = K * K
    blk = min(_BLK, B)
    nb = B // blk

    latT = latent.T
    seqT = seq.reshape(B, K * D).T
    noiseT = noise_u.reshape(S * B, KK).T
    WsT = W_sink.T
    WmT = W_mask.T
    bs2 = b_sink.reshape(KK, 1)
    bm2 = b_mask.reshape(K, 1)

    noise_specs = [
        pl.BlockSpec((KK, blk), lambda i, s=s: (0, s * nb + i))
        for s in range(S)
    ]

    ordT3, stoppingT = pl.pallas_call(
        _make_body(S, K, D),
        grid=(nb,),
        in_specs=[
            pl.BlockSpec((DL, blk), lambda i: (0, i)),
            pl.BlockSpec((K * D, blk), lambda i: (0, i)),
            *noise_specs,
            pl.BlockSpec((KK, DL), lambda i: (0, 0)),
            pl.BlockSpec((KK, 1), lambda i: (0, 0)),
            pl.BlockSpec((K, DL), lambda i: (0, 0)),
            pl.BlockSpec((K, 1), lambda i: (0, 0)),
        ],
        out_specs=[
            pl.BlockSpec((S, K * D, blk), lambda i: (0, 0, i)),
            pl.BlockSpec((K, blk), lambda i: (0, i)),
        ],
        out_shape=[
            jax.ShapeDtypeStruct((S, K * D, B), jnp.float32),
            jax.ShapeDtypeStruct((K, B), jnp.float32),
        ],
        compiler_params=pltpu.CompilerParams(
            dimension_semantics=("parallel",),
        ),
        name="sinkhorn_net",
    )(latT, seqT, *([noiseT] * S), WsT, bs2, WmT, bm2)

    ordered = ordT3.transpose(0, 2, 1).reshape(S * B, K, D)
    return ordered, stoppingT.T


# speedup vs baseline: 6.0787x; 1.0344x over previous
import jax
import jax.numpy as jnp
from jax.experimental import pallas as pl
from jax.experimental.pallas import tpu as pltpu

_EPS = 1e-20
_N_ITERS = 5


def _make_body(S, K, D, DL):
    KK = K * K
    KD = K * D

    def _body(latX_ref, seqX_ref, *rest):
        noise_refs = rest[:S]
        ws_ref, bs_ref, wm_ref, bm_ref, ord_ref, stop_ref = rest[S:]

        lat = [latX_ref[dl, 0] for dl in range(DL)]

        a0 = []
        for rj in range(KK):
            acc = lat[0] * ws_ref[0, rj]
            for dl in range(1, DL):
                acc = acc + lat[dl] * ws_ref[dl, rj]
            a0.append(jnp.exp(acc + bs_ref[rj]))

        st = []
        for k in range(K):
            acc = lat[0] * wm_ref[0, k]
            for dl in range(1, DL):
                acc = acc + lat[dl] * wm_ref[dl, k]
            st.append(acc + bm_ref[k])
        m = st[0]
        for k in range(1, K):
            m = jnp.maximum(m, st[k])
        e = [jnp.exp(x - m) for x in st]
        ssum = e[0]
        for k in range(1, K):
            ssum = ssum + e[k]
        rs = 1.0 / ssum
        for k in range(K):
            stop_ref[k, 0] = e[k] * rs

        seqv = [seqX_ref[jd, 0] for jd in range(KD)]

        for s in range(S):
            a = []
            for r in range(K):
                arow = []
                for j in range(K):
                    u = noise_refs[s][K * r + j, 0, 0]
                    denom = _EPS - jnp.log(u + _EPS)
                    arow.append(a0[K * r + j] / denom)
                a.append(arow)
            for _ in range(_N_ITERS):
                for r in range(K):
                    t = a[r][0]
                    for j in range(1, K):
                        t = t + a[r][j]
                    rr = 1.0 / t
                    a[r] = [x * rr for x in a[r]]
                for j in range(K):
                    t = a[0][j]
                    for r in range(1, K):
                        t = t + a[r][j]
                    rc = 1.0 / t
                    for r in range(K):
                        a[r][j] = a[r][j] * rc
            for i in range(K):
                for d in range(D):
                    acc = a[0][i] * seqv[d]
                    for j in range(1, K):
                        acc = acc + a[j][i] * seqv[D * j + d]
                    ord_ref[s, D * i + d, 0] = acc

    return _body


def kernel(latent, seq, noise_u, W_sink, b_sink, W_mask, b_mask):
    B, DL = latent.shape
    _, K, D = seq.shape
    S = noise_u.shape[0] // B
    KK = K * K
    KD = K * D
    nb = B // 1024

    latX = latent.T.reshape(DL, nb, 8, 128)
    seqX = seq.reshape(B, KD).T.reshape(KD, nb, 8, 128)
    noiseX = noise_u.reshape(S * B, KK).T.reshape(KK, S, nb, 8, 128)

    noise_specs = [
        pl.BlockSpec((KK, 1, 1, 8, 128), lambda i, s=s: (0, s, i, 0, 0))
        for s in range(S)
    ]

    ordX, stopX = pl.pallas_call(
        _make_body(S, K, D, DL),
        grid=(nb,),
        in_specs=[
            pl.BlockSpec((DL, 1, 8, 128), lambda i: (0, i, 0, 0)),
            pl.BlockSpec((KD, 1, 8, 128), lambda i: (0, i, 0, 0)),
            *noise_specs,
            pl.BlockSpec(memory_space=pltpu.SMEM),
            pl.BlockSpec(memory_space=pltpu.SMEM),
            pl.BlockSpec(memory_space=pltpu.SMEM),
            pl.BlockSpec(memory_space=pltpu.SMEM),
        ],
        out_specs=[
            pl.BlockSpec((S, KD, 1, 8, 128), lambda i: (0, 0, i, 0, 0)),
            pl.BlockSpec((K, 1, 8, 128), lambda i: (0, i, 0, 0)),
        ],
        out_shape=[
            jax.ShapeDtypeStruct((S, KD, nb, 8, 128), jnp.float32),
            jax.ShapeDtypeStruct((K, nb, 8, 128), jnp.float32),
        ],
        compiler_params=pltpu.CompilerParams(
            dimension_semantics=("parallel",),
        ),
        name="sinkhorn_net",
    )(latX, seqX, *([noiseX] * S), W_sink, b_sink, W_mask, b_mask)

    ordered = ordX.reshape(S, KD, B).transpose(0, 2, 1).reshape(S * B, K, D)
    return ordered, stopX.reshape(K, B).T


# speedup vs baseline: 6.7121x; 1.1042x over previous
import jax
import jax.numpy as jnp
from jax.experimental import pallas as pl
from jax.experimental.pallas import tpu as pltpu

_EPS = 1e-20
_N_ITERS = 5
_N_CHUNKS = 4


def _make_body(S, K, D, DL):
    KK = K * K
    KD = K * D

    def _body(latX_ref, seqX_ref, *rest):
        noise_refs = rest[:S]
        ws_ref, bs_ref, wm_ref, bm_ref, ord_ref, stop_ref = rest[S:]

        lat = [latX_ref[dl, 0] for dl in range(DL)]

        a0 = []
        for rj in range(KK):
            acc = lat[0] * ws_ref[0, rj]
            for dl in range(1, DL):
                acc = acc + lat[dl] * ws_ref[dl, rj]
            a0.append(jnp.exp(acc + bs_ref[rj]))

        st = []
        for k in range(K):
            acc = lat[0] * wm_ref[0, k]
            for dl in range(1, DL):
                acc = acc + lat[dl] * wm_ref[dl, k]
            st.append(acc + bm_ref[k])
        m = st[0]
        for k in range(1, K):
            m = jnp.maximum(m, st[k])
        e = [jnp.exp(x - m) for x in st]
        ssum = e[0]
        for k in range(1, K):
            ssum = ssum + e[k]
        rs = 1.0 / ssum
        for k in range(K):
            stop_ref[k, 0] = e[k] * rs

        seqv = [seqX_ref[jd, 0] for jd in range(KD)]

        for s in range(S):
            a = []
            for r in range(K):
                arow = []
                for j in range(K):
                    u = noise_refs[s][K * r + j, 0, 0]
                    denom = _EPS - jnp.log(u + _EPS)
                    arow.append(a0[K * r + j] / denom)
                a.append(arow)
            for _ in range(_N_ITERS):
                for r in range(K):
                    t = a[r][0]
                    for j in range(1, K):
                        t = t + a[r][j]
                    rr = 1.0 / t
                    a[r] = [x * rr for x in a[r]]
                for j in range(K):
                    t = a[0][j]
                    for r in range(1, K):
                        t = t + a[r][j]
                    rc = 1.0 / t
                    for r in range(K):
                        a[r][j] = a[r][j] * rc
            for i in range(K):
                for d in range(D):
                    acc = a[0][i] * seqv[d]
                    for j in range(1, K):
                        acc = acc + a[j][i] * seqv[D * j + d]
                    ord_ref[s, D * i + d, 0] = acc

    return _body


def kernel(latent, seq, noise_u, W_sink, b_sink, W_mask, b_mask):
    B, DL = latent.shape
    _, K, D = seq.shape
    S = noise_u.shape[0] // B
    KK = K * K
    KD = K * D

    nc = _N_CHUNKS if B % (_N_CHUNKS * 1024) == 0 else 1
    Bc = B // nc
    nb = Bc // 1024

    seq2 = seq.reshape(B, KD)
    noise3 = noise_u.reshape(S, B, KK)

    body = _make_body(S, K, D, DL)
    noise_specs = [
        pl.BlockSpec((KK, 1, 1, 8, 128), lambda i, s=s: (0, s, i, 0, 0))
        for s in range(S)
    ]

    ord_pieces = []
    stop_pieces = []
    for c in range(nc):
        sl = slice(c * Bc, (c + 1) * Bc)
        latX = latent[sl].T.reshape(DL, nb, 8, 128)
        seqX = seq2[sl].T.reshape(KD, nb, 8, 128)
        noiseX = noise3[:, sl].transpose(2, 0, 1).reshape(KK, S, nb, 8, 128)

        ordX, stopX = pl.pallas_call(
            body,
            grid=(nb,),
            in_specs=[
                pl.BlockSpec((DL, 1, 8, 128), lambda i: (0, i, 0, 0)),
                pl.BlockSpec((KD, 1, 8, 128), lambda i: (0, i, 0, 0)),
                *noise_specs,
                pl.BlockSpec(memory_space=pltpu.SMEM),
                pl.BlockSpec(memory_space=pltpu.SMEM),
                pl.BlockSpec(memory_space=pltpu.SMEM),
                pl.BlockSpec(memory_space=pltpu.SMEM),
            ],
            out_specs=[
                pl.BlockSpec((S, KD, 1, 8, 128), lambda i: (0, 0, i, 0, 0)),
                pl.BlockSpec((K, 1, 8, 128), lambda i: (0, i, 0, 0)),
            ],
            out_shape=[
                jax.ShapeDtypeStruct((S, KD, nb, 8, 128), jnp.float32),
                jax.ShapeDtypeStruct((K, nb, 8, 128), jnp.float32),
            ],
            compiler_params=pltpu.CompilerParams(
                dimension_semantics=("parallel",),
            ),
            name="sinkhorn_net",
        )(latX, seqX, *([noiseX] * S), W_sink, b_sink, W_mask, b_mask)

        ord_pieces.append(
            ordX.reshape(S, KD, Bc).transpose(0, 2, 1).reshape(S, Bc, K, D))
        stop_pieces.append(stopX.reshape(K, Bc).T)

    ordered = jnp.concatenate(ord_pieces, axis=1).reshape(S * B, K, D)
    stopping = jnp.concatenate(stop_pieces, axis=0)
    return ordered, stopping
